# baseline (speedup 1.0000x reference)
import jax
import jax.numpy as jnp
from jax.experimental import pallas as pl
from jax.experimental.pallas import tpu as pltpu

_CHUNKS = 4


def _rnn_body(sseq_ref, seq_ref, xa_ref, xb_ref, wx0_ref, wh0_ref, wcat_ref,
              b0_ref, b1_ref, init_ref, out_ref, carry_ref, *, seqlen):
    B = xa_ref.shape[0]
    H = xa_ref.shape[2]
    CS = xa_ref.shape[1]
    CB = CS // 8
    NC = seqlen // CS
    c = pl.program_id(0)
    tc0 = c * CS

    wh0 = wh0_ref[0]
    wcat = wcat_ref[0]
    b0 = b0_ref[...]
    b1 = b1_ref[...]
    seq = seq_ref[...]
    init = jnp.broadcast_to(init_ref[...], (B, H))
    zero = jnp.zeros((B, H), jnp.float32)

    m = sseq_ref[0, 0]
    for bi in range(1, B):
        m = jnp.maximum(m, sseq_ref[bi, 0])
    jactive = m // 8

    def bdot(a, w):
        return jnp.dot(a.astype(jnp.bfloat16), w,
                       preferred_element_type=jnp.float32)

    def project(src_ref, jl):
        xs = src_ref[:, pl.ds(jl * 8, 8), :]
        pr = jax.lax.dot_general(
            xs, wx0_ref[0], (((2,), (0,)), ((), ())),
            preferred_element_type=jnp.float32) + b0[:, None, :]
        prT = jnp.swapaxes(pr, 0, 1)
        return [prT[i] for i in range(8)]

    def flush(jl, buf, lane):
        blk = jnp.swapaxes(jnp.stack(buf, 0), 0, 1)
        out_ref[:, pl.ds(jl * 8, 8), lane * H:(lane + 1) * H] = blk

    HB = B // 2
    seqh = [seq[:HB], seq[HB:]]

    def rstep_h(t, ps, xp_t, half):
        p0, p1 = ps
        h0 = jnp.tanh(p0 + xp_t)
        h1 = jnp.tanh(p1 + b1)
        sq = seqh[half]
        h0m = jnp.where(sq > t, h0, 0.0)
        h1m = jnp.where(sq > (t - 1), h1, 0.0)
        np0 = bdot(h0, wh0)
        np1 = bdot(jnp.concatenate([h0, h1], axis=1), wcat)
        return (np0, np1), h0m, h1m

    def rstep(t, ps, xp_t):
        (p0a, p0b), (p1a, p1b) = ps
        psa, h0ma, h1ma = rstep_h(t, (p0a, p1a), xp_t[:HB], 0)
        psb, h0mb, h1mb = rstep_h(t, (p0b, p1b), xp_t[HB:], 1)
        ps = ((psa[0], psb[0]), (psa[1], psb[1]))
        h0m = jnp.concatenate([h0ma, h0mb], axis=0)
        h1m = jnp.concatenate([h1ma, h1mb], axis=0)
        return ps, h0m, h1m

    def split_ps(p0, p1):
        return ((p0[:HB], p0[HB:]), (p1[:HB], p1[HB:]))

    def join_ps(ps):
        return (jnp.concatenate(ps[0], axis=0), jnp.concatenate(ps[1], axis=0))

    def store_carry(p0, p1, xp, h1buf):
        vals = [p0, p1] + list(xp) + list(h1buf)
        for i, v in enumerate(vals):
            carry_ref[i] = v

    def load_carry():
        return (carry_ref[0], carry_ref[1],
                [carry_ref[2 + i] for i in range(8)],
                [carry_ref[10 + i] for i in range(7)])

    @pl.when(c == 0)
    def _():
        xp = project(xa_ref, 0)
        p0 = bdot(init, wh0)
        p1 = bdot(jnp.concatenate([init, init], axis=1), wcat)
        h0 = jnp.tanh(p0 + xp[0])
        h0buf = [jnp.where(seq > 0, h0, 0.0)]
        h1buf = []
        ps = split_ps(bdot(h0, wh0),
                      bdot(jnp.concatenate([h0, init], axis=1), wcat))
        for i in range(1, 8):
            ps, h0m, h1m = rstep(i, ps, xp[i])
            h0buf.append(h0m)
            h1buf.append(h1m)
        flush(0, h0buf, 0)
        store_carry(*join_ps(ps), project(xa_ref, 1), h1buf)

    @pl.when(c > 0)
    def _():
        p0 = carry_ref[0]
        h1last = carry_ref[1]
        xp = [carry_ref[2 + i] for i in range(8)]

        def act(_):
            h0 = jnp.tanh(p0 + xp[0])
            h0buf = [jnp.where(seq > tc0, h0, 0.0)]
            h1buf = []
            ps = split_ps(bdot(h0, wh0),
                          bdot(jnp.concatenate([h0, h1last], axis=1), wcat))
            for i in range(1, 8):
                ps, h0m, h1m = rstep(tc0 + i, ps, xp[i])
                h0buf.append(h0m)
                h1buf.append(h1m)
            flush(0, h0buf, 0)
            store_carry(*join_ps(ps), project(xa_ref, 1), h1buf)
            return 0

        def inact(_):
            out_ref[:, pl.ds(0, 8), 0:H] = jnp.zeros((B, 8, H), jnp.float32)
            store_carry(p0, p0, xp, [zero] * 7)
            return 0

        jax.lax.cond(c * CB <= jactive, act, inact, 0)

    p0, p1, xp, h1buf = load_carry()

    def make_block(project_next):
        def block(jl, carry):
            def act(carry):
                ps = split_ps(carry[0], carry[1])
                xp = list(carry[2:10])
                h1buf = list(carry[10:17])
                t0 = tc0 + jl * 8
                ps, h0m, h1m = rstep(t0, ps, xp[0])
                flush(jl - 1, h1buf + [h1m], 1)
                h0buf = [h0m]
                h1buf = []
                for i in range(1, 8):
                    ps, h0m, h1m = rstep(t0 + i, ps, xp[i])
                    h0buf.append(h0m)
                    h1buf.append(h1m)
                flush(jl, h0buf, 0)
                return (*join_ps(ps), *project_next(jl), *h1buf)

            def inact(carry):
                h1buf = list(carry[10:17])
                flush(jl - 1, h1buf + [zero], 1)
                out_ref[:, pl.ds(jl * 8, 8), 0:H] = jnp.zeros(
                    (B, 8, H), jnp.float32)
                return carry[:10] + (zero,) * 7

            return jax.lax.cond(c * CB + jl <= jactive, act, inact, carry)
        return block

    carry = (p0, p1, *xp, *h1buf)
    carry = jax.lax.fori_loop(
        1, CB - 1, make_block(lambda jl: project(xa_ref, jl + 1)), carry,
        unroll=1)
    carry = make_block(lambda jl: project(xb_ref, 0))(CB - 1, carry)

    ps = carry[:2]
    xp_next = list(carry[2:10])
    h1buf = list(carry[10:17])
    t_last = tc0 + CS - 1
    h1 = jnp.tanh(ps[1] + b1)
    h1buf.append(jnp.where(seq > t_last, h1, 0.0))
    flush(CB - 1, h1buf, 1)
    store_carry(ps[0], h1, xp_next, [zero] * 7)


def kernel(input, seq_lens, W_x, W_h, b, init_state, batch_size, depth, output_size):
    B, S, H = input.shape
    DEPTH = W_x.shape[0]
    NC = _CHUNKS
    CS = S // NC

    seq2d = seq_lens.reshape(B, 1)
    wh0 = W_h[0:1].astype(jnp.bfloat16)
    wcat = jnp.concatenate([W_x[1:2], W_h[1:2]],
                           axis=1).astype(jnp.bfloat16)
    b0 = b[0].reshape(1, H)
    b1 = b[1].reshape(1, H)

    out = pl.pallas_call(
        lambda *refs: _rnn_body(*refs, seqlen=S),
        grid=(NC,),
        in_specs=[
            pl.BlockSpec(memory_space=pltpu.SMEM),
            pl.BlockSpec((B, 1), lambda c: (0, 0)),
            pl.BlockSpec((B, CS, H), lambda c: (0, c, 0)),
            pl.BlockSpec((B, CS, H),
                         lambda c: (0, jnp.minimum(c + 1, NC - 1), 0)),
            pl.BlockSpec((1, H, H), lambda c: (0, 0, 0)),
            pl.BlockSpec((1, H, H), lambda c: (0, 0, 0)),
            pl.BlockSpec((1, 2 * H, H), lambda c: (0, 0, 0)),
            pl.BlockSpec((1, H), lambda c: (0, 0)),
            pl.BlockSpec((1, H), lambda c: (0, 0)),
            pl.BlockSpec((1, H), lambda c: (0, 0)),
        ],
        out_specs=pl.BlockSpec((B, CS, DEPTH * H), lambda c: (0, c, 0)),
        out_shape=jax.ShapeDtypeStruct((B, S, DEPTH * H), jnp.float32),
        scratch_shapes=[pltpu.VMEM((17, B, H), jnp.float32)],
    )(seq2d, seq2d, input, input, W_x[0][None], wh0, wcat, b0, b1, init_state)

    return out.reshape(B, S, DEPTH, H)


# speedup vs baseline: 1.1229x; 1.1229x over previous
import jax
import jax.numpy as jnp
from jax.experimental import pallas as pl
from jax.experimental.pallas import tpu as pltpu

_CHUNKS = 8


def _rnn_body(sseq_ref, seq_ref, xa_ref, xb_ref, wx0_ref, wh0_ref, wcat_ref,
              b0_ref, b1_ref, init_ref, out_ref, carry_ref, *, seqlen):
    B = xa_ref.shape[0]
    H = xa_ref.shape[2]
    CS = xa_ref.shape[1]
    CB = CS // 8
    NC = seqlen // CS
    c = pl.program_id(0)
    tc0 = c * CS

    wh0 = wh0_ref[0]
    wcat = wcat_ref[0]
    b0 = b0_ref[...]
    b1 = b1_ref[...]
    seq = seq_ref[...]
    init = jnp.broadcast_to(init_ref[...], (B, H))
    zero = jnp.zeros((B, H), jnp.float32)

    m = sseq_ref[0, 0]
    for bi in range(1, B):
        m = jnp.maximum(m, sseq_ref[bi, 0])
    jactive = m // 8

    def bdot(a, w):
        return jnp.dot(a.astype(jnp.bfloat16), w,
                       preferred_element_type=jnp.float32)

    def project(src_ref, jl):
        xs = src_ref[:, pl.ds(jl * 8, 8), :]
        pr = jax.lax.dot_general(
            xs, wx0_ref[0], (((2,), (0,)), ((), ())),
            preferred_element_type=jnp.float32) + b0[:, None, :]
        prT = jnp.swapaxes(pr, 0, 1)
        return [prT[i] for i in range(8)]

    def flush(jl, buf, lane):
        blk = jnp.swapaxes(jnp.stack(buf, 0), 0, 1)
        out_ref[:, pl.ds(jl * 8, 8), lane * H:(lane + 1) * H] = blk

    def rstep(t, ps, xp_t):
        p0, p1 = ps
        h0 = jnp.tanh(p0 + xp_t)
        h1 = jnp.tanh(p1 + b1)
        h0m = jnp.where(seq > t, h0, 0.0)
        h1m = jnp.where(seq > (t - 1), h1, 0.0)
        np0 = bdot(h0, wh0)
        np1 = bdot(jnp.concatenate([h0, h1], axis=1), wcat)
        return (np0, np1), h0m, h1m

    def store_carry(p0, p1, xp, h1buf):
        vals = [p0, p1] + list(xp) + list(h1buf)
        for i, v in enumerate(vals):
            carry_ref[i] = v

    def load_carry():
        return (carry_ref[0], carry_ref[1],
                [carry_ref[2 + i] for i in range(8)],
                [carry_ref[10 + i] for i in range(7)])

    @pl.when(c == 0)
    def _():
        xp = project(xa_ref, 0)
        p0 = bdot(init, wh0)
        p1 = bdot(jnp.concatenate([init, init], axis=1), wcat)
        h0 = jnp.tanh(p0 + xp[0])
        h0buf = [jnp.where(seq > 0, h0, 0.0)]
        h1buf = []
        ps = (bdot(h0, wh0), bdot(jnp.concatenate([h0, init], axis=1), wcat))
        for i in range(1, 8):
            ps, h0m, h1m = rstep(i, ps, xp[i])
            h0buf.append(h0m)
            h1buf.append(h1m)
        flush(0, h0buf, 0)
        store_carry(*ps, project(xa_ref, 1), h1buf)

    @pl.when(c > 0)
    def _():
        p0 = carry_ref[0]
        h1last = carry_ref[1]
        xp = [carry_ref[2 + i] for i in range(8)]

        def act(_):
            h0 = jnp.tanh(p0 + xp[0])
            h0buf = [jnp.where(seq > tc0, h0, 0.0)]
            h1buf = []
            ps = (bdot(h0, wh0),
                  bdot(jnp.concatenate([h0, h1last], axis=1), wcat))
            for i in range(1, 8):
                ps, h0m, h1m = rstep(tc0 + i, ps, xp[i])
                h0buf.append(h0m)
                h1buf.append(h1m)
            flush(0, h0buf, 0)
            store_carry(*ps, project(xa_ref, 1), h1buf)
            return 0

        def inact(_):
            out_ref[:, pl.ds(0, 8), 0:H] = jnp.zeros((B, 8, H), jnp.float32)
            store_carry(p0, p0, xp, [zero] * 7)
            return 0

        jax.lax.cond(c * CB <= jactive, act, inact, 0)

    p0, p1, xp, h1buf = load_carry()

    def make_block(project_next):
        def block(jl, carry):
            def act(carry):
                ps = carry[:2]
                xp = list(carry[2:10])
                h1buf = list(carry[10:17])
                t0 = tc0 + jl * 8
                ps, h0m, h1m = rstep(t0, ps, xp[0])
                flush(jl - 1, h1buf + [h1m], 1)
                h0buf = [h0m]
                h1buf = []
                for i in range(1, 8):
                    ps, h0m, h1m = rstep(t0 + i, ps, xp[i])
                    h0buf.append(h0m)
                    h1buf.append(h1m)
                flush(jl, h0buf, 0)
                return (*ps, *project_next(jl), *h1buf)

            def inact(carry):
                h1buf = list(carry[10:17])
                flush(jl - 1, h1buf + [zero], 1)
                out_ref[:, pl.ds(jl * 8, 8), 0:H] = jnp.zeros(
                    (B, 8, H), jnp.float32)
                return carry[:10] + (zero,) * 7

            return jax.lax.cond(c * CB + jl <= jactive, act, inact, carry)
        return block

    carry = (p0, p1, *xp, *h1buf)
    carry = jax.lax.fori_loop(
        1, CB - 1, make_block(lambda jl: project(xa_ref, jl + 1)), carry,
        unroll=1)
    carry = make_block(lambda jl: project(xb_ref, 0))(CB - 1, carry)

    ps = carry[:2]
    xp_next = list(carry[2:10])
    h1buf = list(carry[10:17])
    t_last = tc0 + CS - 1
    h1 = jnp.tanh(ps[1] + b1)
    h1buf.append(jnp.where(seq > t_last, h1, 0.0))
    flush(CB - 1, h1buf, 1)
    store_carry(ps[0], h1, xp_next, [zero] * 7)


def kernel(input, seq_lens, W_x, W_h, b, init_state, batch_size, depth, output_size):
    B, S, H = input.shape
    DEPTH = W_x.shape[0]
    NC = _CHUNKS
    CS = S // NC

    seq2d = seq_lens.reshape(B, 1)
    wh0 = W_h[0:1].astype(jnp.bfloat16)
    wcat = jnp.concatenate([W_x[1:2], W_h[1:2]],
                           axis=1).astype(jnp.bfloat16)
    b0 = b[0].reshape(1, H)
    b1 = b[1].reshape(1, H)

    out = pl.pallas_call(
        lambda *refs: _rnn_body(*refs, seqlen=S),
        grid=(NC,),
        in_specs=[
            pl.BlockSpec(memory_space=pltpu.SMEM),
            pl.BlockSpec((B, 1), lambda c: (0, 0)),
            pl.BlockSpec((B, CS, H), lambda c: (0, c, 0)),
            pl.BlockSpec((B, CS, H),
                         lambda c: (0, jnp.minimum(c + 1, NC - 1), 0)),
            pl.BlockSpec((1, H, H), lambda c: (0, 0, 0)),
            pl.BlockSpec((1, H, H), lambda c: (0, 0, 0)),
            pl.BlockSpec((1, 2 * H, H), lambda c: (0, 0, 0)),
            pl.BlockSpec((1, H), lambda c: (0, 0)),
            pl.BlockSpec((1, H), lambda c: (0, 0)),
            pl.BlockSpec((1, H), lambda c: (0, 0)),
        ],
        out_specs=pl.BlockSpec((B, CS, DEPTH * H), lambda c: (0, c, 0)),
        out_shape=jax.ShapeDtypeStruct((B, S, DEPTH * H), jnp.float32),
        scratch_shapes=[pltpu.VMEM((17, B, H), jnp.float32)],
    )(seq2d, seq2d, input, input, W_x[0][None], wh0, wcat, b0, b1, init_state)

    return out.reshape(B, S, DEPTH, H)


# speedup vs baseline: 1.1258x; 1.0026x over previous
import jax
import jax.numpy as jnp
from jax.experimental import pallas as pl
from jax.experimental.pallas import tpu as pltpu

_CHUNKS = 8


def _rnn_body(sseq_ref, seq_ref, xa_ref, xb_ref, wx0_ref, wh0_ref, wcat_ref,
              b0_ref, b1_ref, init_ref, out_ref, carry_ref, *, seqlen):
    B = xa_ref.shape[0]
    H = xa_ref.shape[2]
    CS = xa_ref.shape[1]
    CB = CS // 8
    NC = seqlen // CS
    c = pl.program_id(0)
    tc0 = c * CS

    wh0 = wh0_ref[0]
    wcat = wcat_ref[0]
    b0 = b0_ref[...]
    b1 = b1_ref[...]
    seq = seq_ref[...]
    init = jnp.broadcast_to(init_ref[...], (B, H))
    zero = jnp.zeros((B, H), jnp.float32)

    m = sseq_ref[0, 0]
    for bi in range(1, B):
        m = jnp.maximum(m, sseq_ref[bi, 0])
    jactive = m // 8

    def bdot(a, w):
        return jnp.dot(a.astype(jnp.bfloat16), w,
                       preferred_element_type=jnp.float32)

    def project(src_ref, jl):
        xs = src_ref[:, pl.ds(jl * 8, 8), :]
        pr = jax.lax.dot_general(
            xs, wx0_ref[0], (((2,), (0,)), ((), ())),
            preferred_element_type=jnp.float32) + b0[:, None, :]
        prT = jnp.swapaxes(pr, 0, 1)
        return [prT[i] for i in range(8)]

    def flush(jl, buf, lane):
        blk = jnp.swapaxes(jnp.stack(buf, 0), 0, 1)
        out_ref[:, pl.ds(jl * 8, 8), lane * H:(lane + 1) * H] = blk

    def rstep(t, ps, xp_t):
        p0, p1 = ps
        h0 = jnp.tanh(p0 + xp_t)
        h1 = jnp.tanh(p1 + b1)
        h0m = jnp.where(seq > t, h0, 0.0)
        h1m = jnp.where(seq > (t - 1), h1, 0.0)
        np0 = bdot(h0, wh0)
        np1 = bdot(jnp.concatenate([h0, h1], axis=1), wcat)
        return (np0, np1), h0m, h1m

    def store_carry(p0, p1, xp, h1buf):
        vals = [p0, p1] + list(xp) + list(h1buf)
        for i, v in enumerate(vals):
            carry_ref[i] = v

    def load_carry():
        return (carry_ref[0], carry_ref[1],
                [carry_ref[2 + i] for i in range(8)],
                [carry_ref[10 + i] for i in range(7)])

    @pl.when(c == 0)
    def _():
        xp = project(xa_ref, 0)
        p0 = bdot(init, wh0)
        p1 = bdot(jnp.concatenate([init, init], axis=1), wcat)
        h0 = jnp.tanh(p0 + xp[0])
        h0buf = [jnp.where(seq > 0, h0, 0.0)]
        h1buf = []
        ps = (bdot(h0, wh0), bdot(jnp.concatenate([h0, init], axis=1), wcat))
        for i in range(1, 8):
            ps, h0m, h1m = rstep(i, ps, xp[i])
            h0buf.append(h0m)
            h1buf.append(h1m)
        flush(0, h0buf, 0)
        store_carry(*ps, project(xa_ref, 1), h1buf)

    @pl.when(c > 0)
    def _():
        p0 = carry_ref[0]
        h1last = carry_ref[1]
        xp = [carry_ref[2 + i] for i in range(8)]

        def act(_):
            h0 = jnp.tanh(p0 + xp[0])
            h0buf = [jnp.where(seq > tc0, h0, 0.0)]
            h1buf = []
            ps = (bdot(h0, wh0),
                  bdot(jnp.concatenate([h0, h1last], axis=1), wcat))
            for i in range(1, 8):
                ps, h0m, h1m = rstep(tc0 + i, ps, xp[i])
                h0buf.append(h0m)
                h1buf.append(h1m)
            flush(0, h0buf, 0)
            store_carry(*ps, project(xa_ref, 1), h1buf)
            return 0

        def inact(_):
            out_ref[:, pl.ds(0, 8), 0:H] = jnp.zeros((B, 8, H), jnp.float32)
            store_carry(p0, p0, xp, [zero] * 7)
            return 0

        jax.lax.cond(c * CB <= jactive, act, inact, 0)

    p0, p1, xp, h1buf = load_carry()

    def make_block(project_next):
        def block(jl, carry):
            def act(carry):
                ps = carry[:2]
                xp = list(carry[2:10])
                h1buf = list(carry[10:17])
                t0 = tc0 + jl * 8
                ps, h0m, h1m = rstep(t0, ps, xp[0])
                flush(jl - 1, h1buf + [h1m], 1)
                h0buf = [h0m]
                h1buf = []
                for i in range(1, 8):
                    ps, h0m, h1m = rstep(t0 + i, ps, xp[i])
                    h0buf.append(h0m)
                    h1buf.append(h1m)
                flush(jl, h0buf, 0)
                return (*ps, *project_next(jl), *h1buf)

            def inact(carry):
                h1buf = list(carry[10:17])
                flush(jl - 1, h1buf + [zero], 1)
                out_ref[:, pl.ds(jl * 8, 8), 0:H] = jnp.zeros(
                    (B, 8, H), jnp.float32)
                return carry[:10] + (zero,) * 7

            return jax.lax.cond(c * CB + jl <= jactive, act, inact, carry)
        return block

    carry = (p0, p1, *xp, *h1buf)
    carry = jax.lax.fori_loop(
        1, CB - 1, make_block(lambda jl: project(xa_ref, jl + 1)), carry,
        unroll=2)
    carry = make_block(lambda jl: project(xb_ref, 0))(CB - 1, carry)

    ps = carry[:2]
    xp_next = list(carry[2:10])
    h1buf = list(carry[10:17])
    t_last = tc0 + CS - 1
    h1 = jnp.tanh(ps[1] + b1)
    h1buf.append(jnp.where(seq > t_last, h1, 0.0))
    flush(CB - 1, h1buf, 1)
    store_carry(ps[0], h1, xp_next, [zero] * 7)


def kernel(input, seq_lens, W_x, W_h, b, init_state, batch_size, depth, output_size):
    B, S, H = input.shape
    DEPTH = W_x.shape[0]
    NC = _CHUNKS
    CS = S // NC

    seq2d = seq_lens.reshape(B, 1)
    wh0 = W_h[0:1].astype(jnp.bfloat16)
    wcat = jnp.concatenate([W_x[1:2], W_h[1:2]],
                           axis=1).astype(jnp.bfloat16)
    b0 = b[0].reshape(1, H)
    b1 = b[1].reshape(1, H)

    out = pl.pallas_call(
        lambda *refs: _rnn_body(*refs, seqlen=S),
        grid=(NC,),
        in_specs=[
            pl.BlockSpec(memory_space=pltpu.SMEM),
            pl.BlockSpec((B, 1), lambda c: (0, 0)),
            pl.BlockSpec((B, CS, H), lambda c: (0, c, 0)),
            pl.BlockSpec((B, CS, H),
                         lambda c: (0, jnp.minimum(c + 1, NC - 1), 0)),
            pl.BlockSpec((1, H, H), lambda c: (0, 0, 0)),
            pl.BlockSpec((1, H, H), lambda c: (0, 0, 0)),
            pl.BlockSpec((1, 2 * H, H), lambda c: (0, 0, 0)),
            pl.BlockSpec((1, H), lambda c: (0, 0)),
            pl.BlockSpec((1, H), lambda c: (0, 0)),
            pl.BlockSpec((1, H), lambda c: (0, 0)),
        ],
        out_specs=pl.BlockSpec((B, CS, DEPTH * H), lambda c: (0, c, 0)),
        out_shape=jax.ShapeDtypeStruct((B, S, DEPTH * H), jnp.float32),
        scratch_shapes=[pltpu.VMEM((17, B, H), jnp.float32)],
    )(seq2d, seq2d, input, input, W_x[0][None], wh0, wcat, b0, b1, init_state)

    return out.reshape(B, S, DEPTH, H)
